# baseline (speedup 1.0000x reference)
import functools
import math

import jax
import jax.numpy as jnp
from jax.experimental import pallas as pl
from jax.experimental.pallas import tpu as pltpu
from jax.experimental.pallas import tpu_sc as plsc

_SCALE = 64.0
_MARGIN = 0.5
_COS_M = math.cos(_MARGIN)
_SIN_M = math.sin(_MARGIN)
_LOG2E = math.log2(math.e)
_SE = _SCALE * _LOG2E

_R = 8
_CW = 2048


def _sc_gather_rows(logits, labels):
    n_rows, n_cols = logits.shape

    @functools.partial(
        pl.kernel,
        out_type=jax.ShapeDtypeStruct((n_rows, 128), logits.dtype),
        mesh=plsc.ScalarSubcoreMesh(axis_name="c", num_cores=2),
        scratch_types=[
            pltpu.SMEM((n_rows,), jnp.int32),
            pltpu.SemaphoreType.DMA,
            pltpu.SemaphoreType.DMA,
        ],
    )
    def gather_kernel(x_hbm, l_hbm, o_hbm, l_smem, sem_l, sem_d):
        core = jax.lax.axis_index("c")
        pltpu.async_copy(l_hbm, l_smem, sem_l).wait()
        half = n_rows // 2
        base = core * half

        @pl.loop(0, half)
        def _(i):
            r = base + i
            st = (l_smem[r] // 128) * 128
            pltpu.async_copy(x_hbm.at[r, pl.ds(st, 128)], o_hbm.at[r], sem_d)

        @pl.loop(0, half)
        def _(i):
            r = base + i
            st = (l_smem[r] // 128) * 128
            pltpu.make_async_copy(
                x_hbm.at[r, pl.ds(st, 128)], o_hbm.at[r], sem_d
            ).wait()

    return gather_kernel(logits, labels)


def _reorder_rows(a, n_rows, nblk):
    d = a.shape[1]
    b = a.reshape(4, nblk, _R, d)
    return b.transpose(1, 0, 2, 3).reshape(n_rows, d)


def _loss_body(lane_ref, x128_ref, x1_ref, x2_ref, x3_ref, x4_ref,
               out_ref, *, n_rows, n_cols):
    i = pl.program_id(0)
    x_refs = (x1_ref, x2_ref, x3_ref, x4_ref)

    n_full = n_cols // _CW
    tail = n_cols - n_full * _CW

    def tree128(v):
        parts = [v[:, k * 128:(k + 1) * 128] for k in range(v.shape[1] // 128)]
        while len(parts) > 1:
            half = (len(parts) + 1) // 2
            parts = [
                parts[m] + parts[m + half] if m + half < len(parts) else parts[m]
                for m in range(half)
            ]
        return parts[0]

    def col_body(j, accs):
        out = []
        for xr, acc in zip(x_refs, accs):
            xc = jnp.clip(xr[:, pl.ds(j * _CW, _CW)], -1.0, 1.0)
            out.append(acc + tree128(jnp.exp2(xc * _SE - _SE)))
        return tuple(out)

    accs = jax.lax.fori_loop(
        0, n_full, col_body,
        tuple(jnp.zeros((_R, 128), jnp.float32) for _ in x_refs), unroll=2
    )
    s0 = jnp.concatenate(
        [jnp.sum(a, axis=1) for a in accs])
    if tail:
        tails = []
        for xr in x_refs:
            xc = jnp.clip(xr[:, pl.ds(n_full * _CW, tail)], -1.0, 1.0)
            tails.append(jnp.sum(jnp.exp2(xc * _SE - _SE), axis=1))
        s0 = s0 + jnp.concatenate(tails)

    onehot = jax.lax.broadcasted_iota(jnp.int32, (4 * _R, 128), 1) == lane_ref[...]
    c = jnp.sum(jnp.where(onehot, jnp.clip(x128_ref[...], -1.0, 1.0), 0.0), axis=1)

    t_new = _SCALE * (c * _COS_M - _SIN_M * jnp.sqrt(jnp.maximum(1.0 - c * c, 0.0)))
    e_old = jnp.exp2(c * _SE - _SE)
    e_new = jnp.exp(t_new - _SCALE)
    s = s0 - e_old + e_new
    row_loss = _SCALE + jnp.log(s) - t_new

    @pl.when(i == 0)
    def _():
        out_ref[0, 0] = 0.0

    out_ref[0, 0] += jnp.sum(row_loss) * (1.0 / n_rows)


@jax.jit
def kernel(logits, labels):
    n_rows, n_cols = logits.shape
    labels = labels.astype(jnp.int32)

    x128 = _sc_gather_rows(logits, labels)
    lane128 = (labels % 128).reshape(n_rows, 1)

    nblk = n_rows // _R // 4
    row_order = jnp.arange(4 * n_rows // _R)
    row_order = (row_order % 4) * nblk + row_order // 4

    out = pl.pallas_call(
        functools.partial(_loss_body, n_rows=n_rows, n_cols=n_cols),
        grid=(nblk,),
        in_specs=[
            pl.BlockSpec((4 * _R, 1), lambda i: (i, 0)),
            pl.BlockSpec((4 * _R, 128), lambda i: (i, 0)),
            pl.BlockSpec((_R, n_cols), lambda i: (i, 0)),
            pl.BlockSpec((_R, n_cols), lambda i: (i + nblk, 0)),
            pl.BlockSpec((_R, n_cols), lambda i: (i + 2 * nblk, 0)),
            pl.BlockSpec((_R, n_cols), lambda i: (i + 3 * nblk, 0)),
        ],
        out_specs=pl.BlockSpec((1, 1), lambda i: (0, 0), memory_space=pltpu.SMEM),
        out_shape=jax.ShapeDtypeStruct((1, 1), jnp.float32),
    )(_reorder_rows(lane128, n_rows, nblk), _reorder_rows(x128, n_rows, nblk),
      logits, logits, logits, logits)
    return out[0, 0]
